# Initial kernel scaffold; baseline (speedup 1.0000x reference)
import jax
import jax.numpy as jnp
from jax.experimental import pallas as pl


def kernel(interaction_hidden, speaker_ids, target_speaker, interlocutor):
    raise NotImplementedError("write your pallas kernel here")


# speedup vs baseline: 1.9630x; 1.9630x over previous
import functools

import jax
import jax.numpy as jnp
from jax import lax
from jax.experimental import pallas as pl
from jax.experimental.pallas import tpu as pltpu
from jax.experimental.pallas import tpu_sc as plsc

SEQ = 4096
HID = 1024
NUM_TILES = 32
ROWS_PER_TILE = SEQ // NUM_TILES
HALF = ROWS_PER_TILE // 2
ZCH = 16


def _body(hidden_hbm, ids_hbm, t_hbm, q_hbm, out_hbm,
          ids_v, t_v, q_v, dest_v, rows_v, zero_v, sem):
    info = plsc.get_sparse_core_info()
    nc = info.num_cores
    wid = lax.axis_index("s") * nc + lax.axis_index("c")

    pltpu.sync_copy(t_hbm, t_v)
    pltpu.sync_copy(q_hbm, q_v)
    pltpu.sync_copy(ids_hbm, ids_v)
    tv = t_v[pl.ds(0, 16)]
    qv = q_v[pl.ds(0, 16)]

    w8 = wid * 8
    zv16 = jnp.zeros((16,), jnp.int32)

    def count_body(i, carry):
        c0v, c1v = carry
        v = ids_v[pl.ds(i * 16, 16)]
        c0v = c0v + (v == tv).astype(jnp.int32)
        c1v = c1v + (v == qv).astype(jnp.int32)
        return c0v, c1v

    b0v, b1v = lax.fori_loop(0, w8, count_body, (zv16, zv16))
    r0v, r1v = lax.fori_loop(w8 + 8, SEQ // 16, count_body, (zv16, zv16))
    base0 = jnp.sum(b0v)
    base1 = jnp.sum(b1v)

    c0 = base0
    c1 = base1
    for j in range(8):
        v = ids_v[pl.ds((w8 + j) * 16, 16)]
        m0 = v == tv
        m1 = v == qv
        m0i = m0.astype(jnp.int32)
        m1i = m1.astype(jnp.int32)
        cum0 = plsc.cumsum(m0i)
        cum1 = plsc.cumsum(m1i)
        d0 = cum0 + jnp.broadcast_to(c0 - 1, (16,))
        d1 = cum1 + jnp.broadcast_to(SEQ + c1 - 1, (16,))
        dest = jnp.where(m0, d0, d1)
        dest_v[j // 4, pl.ds((j % 4) * 16, 16)] = dest
        c0 = c0 + jnp.sum(m0i)
        c1 = c1 + jnp.sum(m1i)

    count0 = c0 + jnp.sum(r0v)
    count1 = c1 + jnp.sum(r1v)

    for h in range(2):
        pltpu.sync_copy(hidden_hbm.at[pl.ds(wid * ROWS_PER_TILE + h * HALF, HALF)],
                        rows_v)
        pltpu.async_copy(rows_v, out_hbm.at[dest_v.at[h]], sem).wait()

    zvec = jnp.zeros((16,), jnp.float32)
    for r in range(ZCH):
        def zinit(cb, _, r=r):
            zero_v[r, pl.ds(cb * 16, 16)] = zvec
            return 0
        lax.fori_loop(0, HID // 16, zinit, 0)

    for (cnt, end) in ((count0, SEQ), (count1, 2 * SEQ)):
        length = SEQ - cnt
        nch = length // ZCH
        rem = length - nch * ZCH

        def zchunk(k, _, end=end):
            i = wid + k * NUM_TILES
            start = end - (i + 1) * ZCH
            pltpu.sync_copy(zero_v, out_hbm.at[pl.ds(start, ZCH)])
            return 0

        ntrips = (nch - wid + NUM_TILES - 1) // NUM_TILES
        lax.fori_loop(0, ntrips, zchunk, 0)

        @pl.when(wid < rem)
        def _(cnt=cnt, end=end):
            row = (end - SEQ) + cnt + wid
            pltpu.sync_copy(zero_v.at[pl.ds(0, 1)],
                            out_hbm.at[pl.ds(row, 1)])


@jax.jit
def kernel(interaction_hidden, speaker_ids, target_speaker, interlocutor):
    t_arr = jnp.full((16,), target_speaker, jnp.int32)
    q_arr = jnp.full((16,), interlocutor, jnp.int32)
    run = functools.partial(
        pl.kernel,
        mesh=plsc.VectorSubcoreMesh(core_axis_name="c", subcore_axis_name="s"),
        out_type=jax.ShapeDtypeStruct((2 * SEQ, HID), jnp.float32),
        compiler_params=pltpu.CompilerParams(needs_layout_passes=False),
        scratch_types=[
            pltpu.VMEM((SEQ,), jnp.int32),
            pltpu.VMEM((16,), jnp.int32),
            pltpu.VMEM((16,), jnp.int32),
            pltpu.VMEM((2, HALF), jnp.int32),
            pltpu.VMEM((HALF, HID), jnp.float32),
            pltpu.VMEM((ZCH, HID), jnp.float32),
            pltpu.SemaphoreType.DMA,
        ],
    )(_body)
    flat = run(interaction_hidden, speaker_ids.astype(jnp.int32), t_arr, q_arr)
    return flat.reshape(2, SEQ, HID)


# speedup vs baseline: 2.1020x; 1.0708x over previous
import functools

import jax
import jax.numpy as jnp
from jax import lax
from jax.experimental import pallas as pl
from jax.experimental.pallas import tpu as pltpu
from jax.experimental.pallas import tpu_sc as plsc

SEQ = 4096
HID = 1024
NUM_TILES = 32
ROWS_PER_TILE = SEQ // NUM_TILES
CH = 16
NCHUNK = ROWS_PER_TILE // CH
NSLOT = 6
ZCH = 16
NBLK = SEQ // 16


def _body(hidden_hbm, ids_hbm, t_hbm, q_hbm, out_hbm,
          ids_v, t_v, q_v, dest_v, rows_v, zero_v,
          sems_in, sems_out, sem_z):
    info = plsc.get_sparse_core_info()
    nc = info.num_cores
    wid = lax.axis_index("s") * nc + lax.axis_index("c")
    base_row = wid * ROWS_PER_TILE

    for b in range(NSLOT):
        pltpu.async_copy(hidden_hbm.at[pl.ds(base_row + b * CH, CH)],
                         rows_v.at[b], sems_in[b])

    pltpu.sync_copy(t_hbm, t_v)
    pltpu.sync_copy(q_hbm, q_v)
    pltpu.sync_copy(ids_hbm, ids_v)
    tv = t_v[pl.ds(0, 16)]
    qv = q_v[pl.ds(0, 16)]

    zv16 = jnp.zeros((16,), jnp.int32)

    def count_body(i, carry):
        c0v, c1v = carry
        for u in range(8):
            v = ids_v[pl.ds((i * 8 + u) * 16, 16)]
            c0v = c0v + (v == tv).astype(jnp.int32)
            c1v = c1v + (v == qv).astype(jnp.int32)
        return c0v, c1v

    b0v, b1v = lax.fori_loop(0, wid, count_body, (zv16, zv16))
    r0v, r1v = lax.fori_loop(wid + 1, NUM_TILES, count_body, (zv16, zv16))

    c0 = jnp.sum(b0v)
    c1 = jnp.sum(b1v)
    for j in range(8):
        v = ids_v[pl.ds((wid * 8 + j) * 16, 16)]
        m0 = v == tv
        m1 = v == qv
        m0i = m0.astype(jnp.int32)
        m1i = m1.astype(jnp.int32)
        cum0 = plsc.cumsum(m0i)
        cum1 = plsc.cumsum(m1i)
        d0 = cum0 + jnp.broadcast_to(c0 - 1, (16,))
        d1 = cum1 + jnp.broadcast_to(SEQ + c1 - 1, (16,))
        dest_v[j, pl.ds(0, 16)] = jnp.where(m0, d0, d1)
        c0 = c0 + jnp.sum(m0i)
        c1 = c1 + jnp.sum(m1i)

    count0 = c0 + jnp.sum(r0v)
    count1 = c1 + jnp.sum(r1v)

    zvec = jnp.zeros((16,), jnp.float32)
    for r in range(ZCH):
        def zinit(cb, _, r=r):
            zero_v[r, pl.ds(cb * 16, 16)] = zvec
            return 0
        lax.fori_loop(0, HID // 16, zinit, 0)

    ntrips = []
    rems = []
    for (cnt, end) in ((count0, SEQ), (count1, 2 * SEQ)):
        length = SEQ - cnt
        nch = length // ZCH
        rem = length - nch * ZCH

        def zchunk(k, _, end=end):
            i = wid + k * NUM_TILES
            start = end - (i + 1) * ZCH
            pltpu.async_copy(zero_v, out_hbm.at[pl.ds(start, ZCH)], sem_z)
            return 0

        nt = (nch - wid + NUM_TILES - 1) // NUM_TILES
        lax.fori_loop(0, nt, zchunk, 0)
        ntrips.append(nt)
        rems.append(rem)

        @pl.when(wid < rem)
        def _(cnt=cnt, end=end):
            row = (end - SEQ) + cnt + wid
            pltpu.async_copy(zero_v.at[pl.ds(0, 1)],
                             out_hbm.at[pl.ds(row, 1)], sem_z)

    for c in range(NCHUNK):
        b = c % NSLOT
        pltpu.make_async_copy(hidden_hbm.at[pl.ds(base_row + c * CH, CH)],
                              rows_v.at[b], sems_in[b]).wait()
        pltpu.async_copy(rows_v.at[b], out_hbm.at[dest_v.at[c]], sems_out[b])
        if c + NSLOT < NCHUNK:
            pltpu.make_async_copy(rows_v.at[b], out_hbm.at[pl.ds(0, CH)],
                                  sems_out[b]).wait()
            pltpu.async_copy(
                hidden_hbm.at[pl.ds(base_row + (c + NSLOT) * CH, CH)],
                rows_v.at[b], sems_in[b])

    for c in range(max(0, NCHUNK - NSLOT), NCHUNK):
        b = c % NSLOT
        pltpu.make_async_copy(rows_v.at[b], out_hbm.at[pl.ds(0, CH)],
                              sems_out[b]).wait()

    def zdrain(k, _):
        pltpu.make_async_copy(out_hbm.at[pl.ds(0, ZCH)], zero_v, sem_z).wait()
        return 0
    lax.fori_loop(0, ntrips[0] + ntrips[1], zdrain, 0)
    for rem in rems:
        @pl.when(wid < rem)
        def _(rem=rem):
            pltpu.make_async_copy(out_hbm.at[pl.ds(0, 1)],
                                  zero_v.at[pl.ds(0, 1)], sem_z).wait()


@jax.jit
def kernel(interaction_hidden, speaker_ids, target_speaker, interlocutor):
    t_arr = jnp.full((16,), target_speaker, jnp.int32)
    q_arr = jnp.full((16,), interlocutor, jnp.int32)
    run = functools.partial(
        pl.kernel,
        mesh=plsc.VectorSubcoreMesh(core_axis_name="c", subcore_axis_name="s"),
        out_type=jax.ShapeDtypeStruct((2 * SEQ, HID), jnp.float32),
        compiler_params=pltpu.CompilerParams(needs_layout_passes=False),
        scratch_types=[
            pltpu.VMEM((SEQ,), jnp.int32),
            pltpu.VMEM((16,), jnp.int32),
            pltpu.VMEM((16,), jnp.int32),
            pltpu.VMEM((NCHUNK, CH), jnp.int32),
            pltpu.VMEM((NSLOT, CH, HID), jnp.float32),
            pltpu.VMEM((ZCH, HID), jnp.float32),
            [pltpu.SemaphoreType.DMA] * NSLOT,
            [pltpu.SemaphoreType.DMA] * NSLOT,
            pltpu.SemaphoreType.DMA,
        ],
    )(_body)
    flat = run(interaction_hidden, speaker_ids.astype(jnp.int32), t_arr, q_arr)
    return flat.reshape(2, SEQ, HID)


# speedup vs baseline: 2.1473x; 1.0215x over previous
import functools

import jax
import jax.numpy as jnp
from jax import lax
from jax.experimental import pallas as pl
from jax.experimental.pallas import tpu as pltpu
from jax.experimental.pallas import tpu_sc as plsc

SEQ = 4096
HID = 1024
NUM_TILES = 32
ROWS_PER_TILE = SEQ // NUM_TILES
CH = 16
NCHUNK = ROWS_PER_TILE // CH
NSLOT = 6
ZCH = 16


def _body(hidden_hbm, ids_hbm, t_hbm, q_hbm, out_hbm,
          ids_v, t_v, q_v, dest_v, rows_v, zero_v,
          sems_in, sems_out, sem_z, sem_ids):
    info = plsc.get_sparse_core_info()
    nc = info.num_cores
    wid = lax.axis_index("s") * nc + lax.axis_index("c")
    base_row = wid * ROWS_PER_TILE

    for b in range(NSLOT):
        pltpu.async_copy(hidden_hbm.at[pl.ds(base_row + b * CH, CH)],
                         rows_v.at[b], sems_in[b])
    ids_cp = pltpu.make_async_copy(ids_hbm, ids_v, sem_ids)
    ids_cp.start()
    pltpu.sync_copy(t_hbm, t_v)
    pltpu.sync_copy(q_hbm, q_v)

    zvec = jnp.zeros((16,), jnp.float32)
    for r in range(ZCH):
        def zinit(cb, _, r=r):
            zero_v[r, pl.ds(cb * 16, 16)] = zvec
            return 0
        lax.fori_loop(0, HID // 16, zinit, 0)

    ids_cp.wait()
    tv = t_v[pl.ds(0, 16)]
    qv = q_v[pl.ds(0, 16)]
    zv16 = jnp.zeros((16,), jnp.int32)

    def count_body(i, carry):
        c0v, c1v = carry
        for u in range(8):
            v = ids_v[pl.ds((i * 8 + u) * 16, 16)]
            c0v = c0v + (v == tv).astype(jnp.int32)
            c1v = c1v + (v == qv).astype(jnp.int32)
        return c0v, c1v

    b0v, b1v = lax.fori_loop(0, wid, count_body, (zv16, zv16))
    o0v, o1v = count_body(wid, (zv16, zv16))
    r0v, r1v = lax.fori_loop(wid + 1, NUM_TILES, count_body, (zv16, zv16))

    base0 = jnp.sum(b0v)
    base1 = jnp.sum(b1v)
    count0 = base0 + jnp.sum(o0v) + jnp.sum(r0v)
    count1 = base1 + jnp.sum(o1v) + jnp.sum(r1v)

    ntrips = []
    rems = []
    for (cnt, end) in ((count0, SEQ), (count1, 2 * SEQ)):
        length = SEQ - cnt
        nch = length // ZCH
        rem = length - nch * ZCH

        def zchunk(k, _, end=end):
            i = wid + k * NUM_TILES
            start = end - (i + 1) * ZCH
            pltpu.async_copy(zero_v, out_hbm.at[pl.ds(start, ZCH)], sem_z)
            return 0

        nt = (nch - wid + NUM_TILES - 1) // NUM_TILES
        lax.fori_loop(0, nt, zchunk, 0)
        ntrips.append(nt)
        rems.append(rem)

        @pl.when(wid < rem)
        def _(cnt=cnt, end=end):
            row = (end - SEQ) + cnt + wid
            pltpu.async_copy(zero_v.at[pl.ds(0, 1)],
                             out_hbm.at[pl.ds(row, 1)], sem_z)

    c0 = base0
    c1 = base1
    for j in range(8):
        v = ids_v[pl.ds((wid * 8 + j) * 16, 16)]
        m0 = v == tv
        m1 = v == qv
        m0i = m0.astype(jnp.int32)
        m1i = m1.astype(jnp.int32)
        cum0 = plsc.cumsum(m0i)
        cum1 = plsc.cumsum(m1i)
        d0 = cum0 + jnp.broadcast_to(c0 - 1, (16,))
        d1 = cum1 + jnp.broadcast_to(SEQ + c1 - 1, (16,))
        dest_v[j, pl.ds(0, 16)] = jnp.where(m0, d0, d1)
        c0 = c0 + jnp.sum(m0i)
        c1 = c1 + jnp.sum(m1i)

    for c in range(NCHUNK):
        b = c % NSLOT
        pltpu.make_async_copy(hidden_hbm.at[pl.ds(base_row + c * CH, CH)],
                              rows_v.at[b], sems_in[b]).wait()
        pltpu.async_copy(rows_v.at[b], out_hbm.at[dest_v.at[c]], sems_out[b])
        if c + NSLOT < NCHUNK:
            pltpu.make_async_copy(rows_v.at[b], out_hbm.at[pl.ds(0, CH)],
                                  sems_out[b]).wait()
            pltpu.async_copy(
                hidden_hbm.at[pl.ds(base_row + (c + NSLOT) * CH, CH)],
                rows_v.at[b], sems_in[b])

    for c in range(max(0, NCHUNK - NSLOT), NCHUNK):
        b = c % NSLOT
        pltpu.make_async_copy(rows_v.at[b], out_hbm.at[pl.ds(0, CH)],
                              sems_out[b]).wait()

    def zdrain(k, _):
        pltpu.make_async_copy(out_hbm.at[pl.ds(0, ZCH)], zero_v, sem_z).wait()
        return 0
    lax.fori_loop(0, ntrips[0] + ntrips[1], zdrain, 0)
    for rem in rems:
        @pl.when(wid < rem)
        def _(rem=rem):
            pltpu.make_async_copy(out_hbm.at[pl.ds(0, 1)],
                                  zero_v.at[pl.ds(0, 1)], sem_z).wait()


@jax.jit
def kernel(interaction_hidden, speaker_ids, target_speaker, interlocutor):
    t_arr = jnp.full((16,), target_speaker, jnp.int32)
    q_arr = jnp.full((16,), interlocutor, jnp.int32)
    run = functools.partial(
        pl.kernel,
        mesh=plsc.VectorSubcoreMesh(core_axis_name="c", subcore_axis_name="s"),
        out_type=jax.ShapeDtypeStruct((2 * SEQ, HID), jnp.float32),
        compiler_params=pltpu.CompilerParams(needs_layout_passes=False),
        scratch_types=[
            pltpu.VMEM((SEQ,), jnp.int32),
            pltpu.VMEM((16,), jnp.int32),
            pltpu.VMEM((16,), jnp.int32),
            pltpu.VMEM((NCHUNK, CH), jnp.int32),
            pltpu.VMEM((NSLOT, CH, HID), jnp.float32),
            pltpu.VMEM((ZCH, HID), jnp.float32),
            [pltpu.SemaphoreType.DMA] * NSLOT,
            [pltpu.SemaphoreType.DMA] * NSLOT,
            pltpu.SemaphoreType.DMA,
            pltpu.SemaphoreType.DMA,
        ],
    )(_body)
    flat = run(interaction_hidden, speaker_ids.astype(jnp.int32), t_arr, q_arr)
    return flat.reshape(2, SEQ, HID)
